# baseline (speedup 1.0000x reference)
import functools

import jax
import jax.numpy as jnp
from jax import lax
from jax.experimental import pallas as pl
from jax.experimental.pallas import tpu as pltpu
from jax.experimental.pallas import tpu_sc as plsc

GM = 3.5
L = 16
CH = 128
W = 65536
HI = 0xFFFF0000


def _pack(t):
    K, N = t.shape
    npair = K // 2
    nw = -(-N // W)

    def body(in_ref, *out_refs):
        for p in range(npair):
            a = lax.bitcast_convert_type(
                in_ref[2 * p, :].astype(jnp.bfloat16), jnp.uint16)
            b = lax.bitcast_convert_type(
                in_ref[2 * p + 1, :].astype(jnp.bfloat16), jnp.uint16)
            w = a.astype(jnp.uint32) | (b.astype(jnp.uint32) << 16)
            out_refs[p][...] = lax.bitcast_convert_type(w, jnp.int32)

    return pl.pallas_call(
        body,
        grid=(nw,),
        in_specs=[pl.BlockSpec((K, W), lambda w: (0, w))],
        out_specs=[pl.BlockSpec((W,), lambda w: (w,))
                   for _ in range(npair)],
        out_shape=[jax.ShapeDtypeStruct((nw * W,), jnp.int32)
                   for _ in range(npair)],
    )(t)


@functools.partial(jax.jit, static_argnums=(0, 1, 2, 3, 4))
def _biased_mf(B, K, N, NC, NS, user_ids, item_ids, bu, bv, U, V):
    NW = NC * NS
    bpw = B // NW
    nch = bpw // CH
    npair = K // 2

    pu = _pack(U.T)
    pv = _pack(V.T)

    mesh = plsc.VectorSubcoreMesh(core_axis_name="c", subcore_axis_name="s")

    @functools.partial(
        pl.kernel,
        mesh=mesh,
        out_type=jax.ShapeDtypeStruct((B,), jnp.float32),
        compiler_params=pltpu.CompilerParams(needs_layout_passes=False,
                                             use_tc_tiling_on_sc=False),
        scratch_types=[
            pltpu.VMEM((nch, CH), jnp.int32),
            pltpu.VMEM((nch, CH), jnp.int32),
            pltpu.VMEM((npair, nch, CH), jnp.int32),
            pltpu.VMEM((npair, nch, CH), jnp.int32),
            pltpu.VMEM((nch, CH), jnp.float32),
            pltpu.VMEM((nch, CH), jnp.float32),
            pltpu.VMEM((bpw,), jnp.float32),
            pltpu.SemaphoreType.DMA,
        ],
    )
    def k(uids, iids, bu_t, bv_t, *rest):
        pu_t = rest[:npair]
        pv_t = rest[npair:2 * npair]
        out = rest[2 * npair]
        (uidx, iidx, ucol, vcol, bug, bvg, outv, sem) = rest[2 * npair + 1:]
        wid = lax.axis_index("s") * NC + lax.axis_index("c")
        base = wid * bpw

        for c in range(nch):
            pltpu.sync_copy(uids.at[pl.ds(base + c * CH, CH)], uidx.at[c])
            pltpu.sync_copy(iids.at[pl.ds(base + c * CH, CH)], iidx.at[c])

        copies = []
        for c in range(nch):
            copies.append(pltpu.async_copy(
                bu_t.at[uidx.at[c]], bug.at[c], sem))
            copies.append(pltpu.async_copy(
                bv_t.at[iidx.at[c]], bvg.at[c], sem))
            for p in range(npair):
                copies.append(pltpu.async_copy(
                    pu_t[p].at[uidx.at[c]], ucol.at[p, c], sem))
                copies.append(pltpu.async_copy(
                    pv_t[p].at[iidx.at[c]], vcol.at[p, c], sem))
        for cp in copies:
            cp.wait()

        for c in range(nch):
            for o in range(CH // L):
                s = pl.ds(o * L, L)

                def body(p, acc):
                    hi = jnp.full((L,), HI, jnp.uint32)
                    wu = plsc.bitcast(ucol[p, c, s], jnp.uint32)
                    wv = plsc.bitcast(vcol[p, c, s], jnp.uint32)
                    ulo = plsc.bitcast(wu << 16, jnp.float32)
                    vlo = plsc.bitcast(wv << 16, jnp.float32)
                    uhi = plsc.bitcast(wu & hi, jnp.float32)
                    vhi = plsc.bitcast(wv & hi, jnp.float32)
                    return acc + ulo * vlo + uhi * vhi

                acc = lax.fori_loop(0, npair, body,
                                    bug[c, s] + bvg[c, s] + GM)
                outv[pl.ds(c * CH + o * L, L)] = acc

        pltpu.sync_copy(outv, out.at[pl.ds(base, bpw)])

    return k(user_ids, item_ids, bu, bv, *pu, *pv)


def kernel(user_ids, item_ids, bu, bv, U, V):
    B = user_ids.shape[0]
    N, K = U.shape
    info = plsc.get_sparse_core_info()
    return _biased_mf(B, K, N, info.num_cores, info.num_subcores,
                      user_ids.astype(jnp.int32), item_ids.astype(jnp.int32),
                      bu.reshape(-1), bv.reshape(-1), U, V)


# speedup vs baseline: 1.4626x; 1.4626x over previous
import functools

import jax
import jax.numpy as jnp
from jax import lax
from jax.experimental import pallas as pl
from jax.experimental.pallas import tpu as pltpu
from jax.experimental.pallas import tpu_sc as plsc

GM = 3.5
L = 16
CH = 128
W = 65536
HI = 0xFFFF0000


def _pack(t):
    K, N = t.shape
    npair = K // 2
    nw = -(-N // W)

    def body(in_ref, *out_refs):
        hi = jnp.uint32(HI)
        for p in range(npair):
            a = lax.bitcast_convert_type(in_ref[2 * p, :], jnp.uint32)
            b = lax.bitcast_convert_type(in_ref[2 * p + 1, :], jnp.uint32)
            w = (a >> 16) | (b & hi)
            out_refs[p][...] = lax.bitcast_convert_type(w, jnp.int32)

    return pl.pallas_call(
        body,
        grid=(nw,),
        in_specs=[pl.BlockSpec((K, W), lambda w: (0, w))],
        out_specs=[pl.BlockSpec((W,), lambda w: (w,))
                   for _ in range(npair)],
        out_shape=[jax.ShapeDtypeStruct((nw * W,), jnp.int32)
                   for _ in range(npair)],
    )(t)


@functools.partial(jax.jit, static_argnums=(0, 1, 2, 3, 4))
def _biased_mf(B, K, N, NC, NS, user_ids, item_ids, bu, bv, U, V):
    NW = NC * NS
    bpw = B // NW
    nch = bpw // CH
    npair = K // 2

    pu = _pack(U.T)
    pv = _pack(V.T)

    mesh = plsc.VectorSubcoreMesh(core_axis_name="c", subcore_axis_name="s")

    @functools.partial(
        pl.kernel,
        mesh=mesh,
        out_type=jax.ShapeDtypeStruct((B,), jnp.float32),
        compiler_params=pltpu.CompilerParams(needs_layout_passes=False,
                                             use_tc_tiling_on_sc=False),
        scratch_types=[
            pltpu.VMEM((nch, CH), jnp.int32),
            pltpu.VMEM((nch, CH), jnp.int32),
            pltpu.VMEM((npair, nch, CH), jnp.int32),
            pltpu.VMEM((npair, nch, CH), jnp.int32),
            pltpu.VMEM((nch, CH), jnp.float32),
            pltpu.VMEM((nch, CH), jnp.float32),
            pltpu.VMEM((bpw,), jnp.float32),
            pltpu.SemaphoreType.DMA,
        ],
    )
    def k(uids, iids, bu_t, bv_t, *rest):
        pu_t = rest[:npair]
        pv_t = rest[npair:2 * npair]
        out = rest[2 * npair]
        (uidx, iidx, ucol, vcol, bug, bvg, outv, sem) = rest[2 * npair + 1:]
        wid = lax.axis_index("s") * NC + lax.axis_index("c")
        base = wid * bpw

        for c in range(nch):
            pltpu.sync_copy(uids.at[pl.ds(base + c * CH, CH)], uidx.at[c])
            pltpu.sync_copy(iids.at[pl.ds(base + c * CH, CH)], iidx.at[c])

        copies = []
        for c in range(nch):
            copies.append(pltpu.async_copy(
                bu_t.at[uidx.at[c]], bug.at[c], sem))
            copies.append(pltpu.async_copy(
                bv_t.at[iidx.at[c]], bvg.at[c], sem))
            for p in range(npair):
                copies.append(pltpu.async_copy(
                    pu_t[p].at[uidx.at[c]], ucol.at[p, c], sem))
                copies.append(pltpu.async_copy(
                    pv_t[p].at[iidx.at[c]], vcol.at[p, c], sem))
        for cp in copies:
            cp.wait()

        for c in range(nch):
            for o in range(CH // L):
                s = pl.ds(o * L, L)

                def body(p, acc):
                    hi = jnp.full((L,), HI, jnp.uint32)
                    wu = plsc.bitcast(ucol[p, c, s], jnp.uint32)
                    wv = plsc.bitcast(vcol[p, c, s], jnp.uint32)
                    ulo = plsc.bitcast(wu << 16, jnp.float32)
                    vlo = plsc.bitcast(wv << 16, jnp.float32)
                    uhi = plsc.bitcast(wu & hi, jnp.float32)
                    vhi = plsc.bitcast(wv & hi, jnp.float32)
                    return acc + ulo * vlo + uhi * vhi

                acc = lax.fori_loop(0, npair, body,
                                    bug[c, s] + bvg[c, s] + GM)
                outv[pl.ds(c * CH + o * L, L)] = acc

        pltpu.sync_copy(outv, out.at[pl.ds(base, bpw)])

    return k(user_ids, item_ids, bu, bv, *pu, *pv)


def kernel(user_ids, item_ids, bu, bv, U, V):
    B = user_ids.shape[0]
    N, K = U.shape
    info = plsc.get_sparse_core_info()
    return _biased_mf(B, K, N, info.num_cores, info.num_subcores,
                      user_ids.astype(jnp.int32), item_ids.astype(jnp.int32),
                      bu.reshape(-1), bv.reshape(-1), U, V)


# speedup vs baseline: 1.7396x; 1.1894x over previous
import functools

import jax
import jax.numpy as jnp
from jax import lax
from jax.experimental import pallas as pl
from jax.experimental.pallas import tpu as pltpu
from jax.experimental.pallas import tpu_sc as plsc

GM = 3.5
L = 16
CH = 128
RES = 8
W = 262144


def _detile(t):
    K, N = t.shape
    ng = K // RES
    nw = -(-N // W)

    def body(in_ref, *out_refs):
        for r in range(RES):
            out_refs[r][...] = in_ref[r, :]

    return pl.pallas_call(
        body,
        grid=(ng, nw),
        in_specs=[pl.BlockSpec((RES, W), lambda g, w: (g, w))],
        out_specs=[pl.BlockSpec((W,), lambda g, w: (g * nw + w))
                   for _ in range(RES)],
        out_shape=[jax.ShapeDtypeStruct((ng * nw * W,), jnp.float32)
                   for _ in range(RES)],
    )(t)


@functools.partial(jax.jit, static_argnums=(0, 1, 2, 3, 4))
def _biased_mf(B, K, N, NC, NS, user_ids, item_ids, bu, bv, U, V):
    NW = NC * NS
    bpw = B // NW
    nch = bpw // CH
    ngr = K // RES
    NP = -(-N // W) * W

    fu = _detile(U.T)
    fv = _detile(V.T)

    mesh = plsc.VectorSubcoreMesh(core_axis_name="c", subcore_axis_name="s")

    @functools.partial(
        pl.kernel,
        mesh=mesh,
        out_type=jax.ShapeDtypeStruct((B,), jnp.float32),
        compiler_params=pltpu.CompilerParams(needs_layout_passes=False,
                                             use_tc_tiling_on_sc=False),
        scratch_types=[
            pltpu.VMEM((nch, CH), jnp.int32),
            pltpu.VMEM((nch, CH), jnp.int32),
            pltpu.VMEM((ngr, nch, CH), jnp.int32),
            pltpu.VMEM((ngr, nch, CH), jnp.int32),
            pltpu.VMEM((K, nch, CH), jnp.float32),
            pltpu.VMEM((K, nch, CH), jnp.float32),
            pltpu.VMEM((nch, CH), jnp.float32),
            pltpu.VMEM((nch, CH), jnp.float32),
            pltpu.VMEM((bpw,), jnp.float32),
            pltpu.SemaphoreType.DMA,
        ],
    )
    def k(uids, iids, bu_t, bv_t, *rest):
        fu_t = rest[:RES]
        fv_t = rest[RES:2 * RES]
        out = rest[2 * RES]
        (uidx, iidx, ush, ish, ucol, vcol, bug, bvg, outv, sem) = \
            rest[2 * RES + 1:]
        wid = lax.axis_index("s") * NC + lax.axis_index("c")
        base = wid * bpw

        for c in range(nch):
            pltpu.sync_copy(uids.at[pl.ds(base + c * CH, CH)], uidx.at[c])
            pltpu.sync_copy(iids.at[pl.ds(base + c * CH, CH)], iidx.at[c])

        for g in range(ngr):
            for c in range(nch):
                for o in range(CH // L):
                    s = pl.ds(o * L, L)
                    ush[g, c, s] = uidx[c, s] + g * NP
                    ish[g, c, s] = iidx[c, s] + g * NP

        copies = []
        for c in range(nch):
            copies.append(pltpu.async_copy(
                bu_t.at[uidx.at[c]], bug.at[c], sem))
            copies.append(pltpu.async_copy(
                bv_t.at[iidx.at[c]], bvg.at[c], sem))
            for g in range(ngr):
                for r in range(RES):
                    j = g * RES + r
                    copies.append(pltpu.async_copy(
                        fu_t[r].at[ush.at[g, c]], ucol.at[j, c], sem))
                    copies.append(pltpu.async_copy(
                        fv_t[r].at[ish.at[g, c]], vcol.at[j, c], sem))
        for cp in copies:
            cp.wait()

        for c in range(nch):
            for o in range(CH // L):
                s = pl.ds(o * L, L)

                def body(j, acc):
                    return acc + ucol[j, c, s] * vcol[j, c, s]

                acc = lax.fori_loop(0, K, body, bug[c, s] + bvg[c, s] + GM)
                outv[pl.ds(c * CH + o * L, L)] = acc

        pltpu.sync_copy(outv, out.at[pl.ds(base, bpw)])

    return k(user_ids, item_ids, bu, bv, *fu, *fv)


def kernel(user_ids, item_ids, bu, bv, U, V):
    B = user_ids.shape[0]
    N, K = U.shape
    info = plsc.get_sparse_core_info()
    return _biased_mf(B, K, N, info.num_cores, info.num_subcores,
                      user_ids.astype(jnp.int32), item_ids.astype(jnp.int32),
                      bu.reshape(-1), bv.reshape(-1), U, V)


# speedup vs baseline: 1.8364x; 1.0556x over previous
import functools

import jax
import jax.numpy as jnp
from jax import lax
from jax.experimental import pallas as pl
from jax.experimental.pallas import tpu as pltpu
from jax.experimental.pallas import tpu_sc as plsc

GM = 3.5
L = 16
CH = 128
RES = 8
W = 262144

_SC_PARAMS = dict(
    compiler_params=pltpu.CompilerParams(needs_layout_passes=False,
                                         use_tc_tiling_on_sc=False))


def _detile(t):
    K, N = t.shape
    ng = K // RES
    nw = -(-N // W)

    def body(in_ref, *out_refs):
        for r in range(RES):
            out_refs[r][...] = in_ref[r, :]

    return pl.pallas_call(
        body,
        grid=(ng, nw),
        in_specs=[pl.BlockSpec((RES, W), lambda g, w: (g, w))],
        out_specs=[pl.BlockSpec((W,), lambda g, w: (g * nw + w))
                   for _ in range(RES)],
        out_shape=[jax.ShapeDtypeStruct((ng * nw * W,), jnp.float32)
                   for _ in range(RES)],
    )(t)


def _stage_indices(ids, idx, base, nch):
    for c in range(nch):
        pltpu.sync_copy(ids.at[pl.ds(base + c * CH, CH)], idx.at[c])


def _shift_indices(idx, sh, ngr, nch, NP):
    for g in range(ngr):
        for c in range(nch):
            for o in range(CH // L):
                s = pl.ds(o * L, L)
                sh[g, c, s] = idx[c, s] + g * NP


def _fire_column_gathers(flats, sh, col, sem, ngr, nch):
    copies = []
    for c in range(nch):
        for g in range(ngr):
            for r in range(RES):
                j = g * RES + r
                copies.append(pltpu.async_copy(
                    flats[r].at[sh.at[g, c]],
                    col.at[j, pl.ds(c * CH, CH)], sem))
    return copies


@functools.partial(jax.jit, static_argnums=(0, 1, 2, 3, 4))
def _biased_mf(B, K, N, NC, NS, user_ids, item_ids, bu, bv, U, V):
    NW = NC * NS
    bpw = B // NW
    nch = bpw // CH
    ngr = K // RES
    NP = -(-N // W) * W

    mesh = plsc.VectorSubcoreMesh(core_axis_name="c", subcore_axis_name="s")

    @functools.partial(
        pl.kernel,
        mesh=mesh,
        out_type=jax.ShapeDtypeStruct((NW, K, bpw), jnp.float32),
        scratch_types=[
            pltpu.VMEM((nch, CH), jnp.int32),
            pltpu.VMEM((ngr, nch, CH), jnp.int32),
            pltpu.VMEM((K, bpw), jnp.float32),
            pltpu.SemaphoreType.DMA,
        ],
        **_SC_PARAMS,
    )
    def gather_u(uids, *rest):
        fu_t = rest[:RES]
        ug = rest[RES]
        (uidx, ush, ucol, sem) = rest[RES + 1:]
        wid = lax.axis_index("s") * NC + lax.axis_index("c")
        _stage_indices(uids, uidx, wid * bpw, nch)
        _shift_indices(uidx, ush, ngr, nch, NP)
        copies = _fire_column_gathers(fu_t, ush, ucol, sem, ngr, nch)
        for cp in copies:
            cp.wait()
        pltpu.sync_copy(ucol, ug.at[wid])

    @functools.partial(
        pl.kernel,
        mesh=mesh,
        out_type=jax.ShapeDtypeStruct((B,), jnp.float32),
        scratch_types=[
            pltpu.VMEM((nch, CH), jnp.int32),
            pltpu.VMEM((nch, CH), jnp.int32),
            pltpu.VMEM((ngr, nch, CH), jnp.int32),
            pltpu.VMEM((K, bpw), jnp.float32),
            pltpu.VMEM((K, bpw), jnp.float32),
            pltpu.VMEM((nch, CH), jnp.float32),
            pltpu.VMEM((nch, CH), jnp.float32),
            pltpu.VMEM((bpw,), jnp.float32),
            pltpu.SemaphoreType.DMA,
        ],
        **_SC_PARAMS,
    )
    def combine(uids, iids, bu_t, bv_t, ug, *rest):
        fv_t = rest[:RES]
        out = rest[RES]
        (uidx, iidx, ish, ucol, vcol, bug, bvg, outv, sem) = rest[RES + 1:]
        wid = lax.axis_index("s") * NC + lax.axis_index("c")
        base = wid * bpw

        _stage_indices(uids, uidx, base, nch)
        _stage_indices(iids, iidx, base, nch)
        _shift_indices(iidx, ish, ngr, nch, NP)

        stage = pltpu.async_copy(ug.at[wid], ucol, sem)
        copies = []
        for c in range(nch):
            copies.append(pltpu.async_copy(
                bu_t.at[uidx.at[c]], bug.at[c], sem))
            copies.append(pltpu.async_copy(
                bv_t.at[iidx.at[c]], bvg.at[c], sem))
        copies += _fire_column_gathers(fv_t, ish, vcol, sem, ngr, nch)
        stage.wait()
        for cp in copies:
            cp.wait()

        for c in range(nch):
            for o in range(CH // L):
                s = pl.ds(o * L, L)
                off = pl.ds(c * CH + o * L, L)

                def body(j, acc):
                    return acc + ucol[j, off] * vcol[j, off]

                acc = lax.fori_loop(0, K, body, bug[c, s] + bvg[c, s] + GM)
                outv[off] = acc

        pltpu.sync_copy(outv, out.at[pl.ds(base, bpw)])

    fu = _detile(U.T)
    ug = gather_u(user_ids, *fu)
    fv = _detile(V.T)
    return combine(user_ids, item_ids, bu, bv, ug, *fv)


def kernel(user_ids, item_ids, bu, bv, U, V):
    B = user_ids.shape[0]
    N, K = U.shape
    info = plsc.get_sparse_core_info()
    return _biased_mf(B, K, N, info.num_cores, info.num_subcores,
                      user_ids.astype(jnp.int32), item_ids.astype(jnp.int32),
                      bu.reshape(-1), bv.reshape(-1), U, V)
